# baseline (speedup 1.0000x reference)
import functools

import jax
import jax.numpy as jnp
from jax import lax
from jax.experimental import pallas as pl
from jax.experimental.pallas import tpu as pltpu
from jax.experimental.pallas import tpu_sc as plsc

B = 8
H = W = 512
D = 64
NC = 1024
H1 = W1 = 256
HQ = WQ = 128
N = B * HQ * WQ
CHUNK = 1024
NCHUNKS = N // CHUNK

_f32 = jnp.float32


def _shift_axis(y, axis, d):
    if d == 0:
        return y
    zero_shape = list(y.shape)
    zero_shape[axis] = 1
    z = jnp.zeros(zero_shape, y.dtype)
    sl = [slice(None)] * y.ndim
    if d == -1:
        sl[axis] = slice(0, y.shape[axis] - 1)
        return jnp.concatenate([z, y[tuple(sl)]], axis=axis)
    else:
        sl[axis] = slice(1, None)
        return jnp.concatenate([y[tuple(sl)], z], axis=axis)


def _encoder_body(x_ref, g4r_ref, g4ct_ref, w1_ref, b1_ref, w2_ref,
                  b2_ref, o_ref):
    x = x_ref[0]
    hw = HQ * WQ
    xr = [jnp.dot(g4r_ref[m], x, preferred_element_type=_f32)
          for m in range(4)]
    xg = [[jnp.dot(xr[m], g4ct_ref[n], preferred_element_type=_f32)
           for n in range(4)] for m in range(4)]

    z1p = [[None, None], [None, None]]
    for a in range(2):
        for bb in range(2):
            planes = []
            for kh in range(4):
                v = 2 * a + kh - 1
                m, dr = v % 4, (v - (v % 4)) // 4
                for kw in range(4):
                    u = 2 * bb + kw - 1
                    n, dc = u % 4, (u - (u % 4)) // 4
                    planes.append(_shift_axis(
                        _shift_axis(xg[m][n], 0, dr), 1, dc))
            p = jnp.stack(planes).reshape(16, hw)
            z1 = jnp.dot(w1_ref[...], p, preferred_element_type=_f32)
            z1 = jnp.maximum(z1 + b1_ref[...], 0.0)
            z1p[a][bb] = z1.reshape(32, HQ, WQ)

    taps = ((1, -1), (0, 0), (1, 0), (0, 1))
    acc = jnp.zeros((D, hw), _f32)
    for kh in range(4):
        a, dr = taps[kh]
        parts = [_shift_axis(_shift_axis(z1p[a][taps[kw][0]], 1, dr),
                             2, taps[kw][1]) for kw in range(4)]
        tk = jnp.concatenate(parts, axis=0).reshape(128, hw)
        acc += jnp.dot(w2_ref[kh], tk, preferred_element_type=_f32)
    o_ref[0] = jnp.maximum(acc + b2_ref[...], 0.0)


def _vq_body(z_ref, cb_ref, idx_ref, loss_ref):
    i = pl.program_id(0)
    z = z_ref[...]
    cb = cb_ref[...]
    s = jnp.dot(cb * -2.0, z, preferred_element_type=_f32)
    cbn = jnp.sum(cb * cb, axis=1, keepdims=True)
    t = s + cbn
    m = jnp.min(t, axis=0, keepdims=True)
    rows = jax.lax.broadcasted_iota(jnp.int32, t.shape, 0)
    idx = jnp.min(jnp.where(t == m, rows, NC), axis=0, keepdims=True)
    idx_ref[0] = idx
    part = jnp.sum(z * z, keepdims=True) + jnp.sum(m, keepdims=True)

    @pl.when(i == 0)
    def _init():
        loss_ref[...] = jnp.zeros_like(loss_ref)

    loss_ref[...] += part

    @pl.when(i == NCHUNKS - 1)
    def _norm():
        loss_ref[...] *= 1.0 / (N * D)


_NW = 32
_GCH = 1024
_BPW = N // _NW
_DP = 64


def _sc_gather_body(cb_hbm, idx_hbm, out_hbm, idx_v, rows_v, sem):
    wid = lax.axis_index("s") * 2 + lax.axis_index("c")
    base = wid * _BPW
    pltpu.sync_copy(idx_hbm.at[pl.ds(base, _BPW)], idx_v)
    for c in range(_BPW // _GCH):
        off = c * _GCH
        pltpu.async_copy(cb_hbm.at[idx_v.at[pl.ds(off, _GCH)]],
                         rows_v, sem).wait()
        pltpu.sync_copy(rows_v, out_hbm.at[pl.ds(base + off, _GCH)])


def _make_sc_gather():
    return functools.partial(
        pl.kernel,
        mesh=plsc.VectorSubcoreMesh(core_axis_name="c", subcore_axis_name="s"),
        out_type=jax.ShapeDtypeStruct((N, _DP), jnp.float32),
        scratch_types=[
            pltpu.VMEM((_BPW,), jnp.int32),
            pltpu.VMEM((_GCH, _DP), jnp.float32),
            pltpu.SemaphoreType.DMA,
        ],
        compiler_params=pltpu.CompilerParams(use_tc_tiling_on_sc=False),
    )(_sc_gather_body)


_PTAPS = {0: ((1, 0), (3, -1)), 1: ((0, 1), (2, 0))}
_OTAPS = {0: ((1, 0, 0), (3, 1, -1)), 1: ((0, 1, 0), (2, 0, 0)),
          2: ((1, 1, 0), (3, 0, 0)), 3: ((0, 0, 1), (2, 1, 0))}


def _decoder_body(zq_ref, wd_ref, bd1_ref, w5_ref, bd2_ref, o_ref):
    zq = zq_ref[0].reshape(D, HQ, WQ)
    hw = HQ * WQ
    q = [[None, None], [None, None]]
    for py in range(2):
        for px in range(2):
            acc = jnp.zeros((32, hw), _f32)
            for (kh, dy) in _PTAPS[py]:
                for (kw, dx) in _PTAPS[px]:
                    t = _shift_axis(_shift_axis(zq, 1, dy), 2, dx)
                    acc += jnp.dot(wd_ref[kh * 4 + kw], t.reshape(D, hw),
                                   preferred_element_type=_f32)
            d_p = jnp.maximum(acc + bd1_ref[...], 0.0)
            qq = jnp.dot(w5_ref[...], d_p, preferred_element_type=_f32)
            q[py][px] = qq.reshape(16, HQ, WQ)
    for s in range(4):
        for t in range(4):
            acc = jnp.zeros((HQ, WQ), _f32)
            for (kh, rp, dy) in _OTAPS[s]:
                for (kw, cp, dx) in _OTAPS[t]:
                    acc += _shift_axis(
                        _shift_axis(q[rp][cp][kh * 4 + kw], 0, dy), 1, dx)
            o_ref[0, s * 4 + t] = jax.nn.sigmoid(acc + bd2_ref[...])


def kernel(x, enc_w1, enc_b1, enc_w2, enc_b2, codebook,
           dec_w1, dec_b1, dec_w2, dec_b2):
    f = _f32
    xs = x[:, 0]
    r4 = jnp.arange(4, dtype=jnp.int32)
    i128 = jnp.arange(128, dtype=jnp.int32)
    i512 = jnp.arange(512, dtype=jnp.int32)
    g4r = (i512[None, None, :] == 4 * i128[None, :, None]
           + r4[:, None, None]).astype(f)
    g4ct = jnp.swapaxes(g4r, 1, 2)
    w1 = enc_w1.reshape(32, 16)
    b1 = enc_b1.reshape(32, 1)
    w2 = jnp.stack([jnp.concatenate([enc_w2[:, :, k, kw]
                                     for kw in range(4)], axis=1)
                    for k in range(4)])
    b2 = enc_b2.reshape(D, 1)
    z_e = pl.pallas_call(
        _encoder_body,
        grid=(B,),
        in_specs=[
            pl.BlockSpec((1, H, W), lambda b: (b, 0, 0)),
            pl.BlockSpec((4, 128, 512), lambda b: (0, 0, 0)),
            pl.BlockSpec((4, 512, 128), lambda b: (0, 0, 0)),
            pl.BlockSpec((32, 16), lambda b: (0, 0)),
            pl.BlockSpec((32, 1), lambda b: (0, 0)),
            pl.BlockSpec((4, D, 128), lambda b: (0, 0, 0)),
            pl.BlockSpec((D, 1), lambda b: (0, 0)),
        ],
        out_specs=pl.BlockSpec((1, D, HQ * WQ), lambda b: (b, 0, 0)),
        out_shape=jax.ShapeDtypeStruct((B, D, HQ * WQ), f),
    )(xs, g4r, g4ct, w1, b1, w2, b2)

    z_flat = z_e.transpose(1, 0, 2).reshape(D, N)
    idx3, loss = pl.pallas_call(
        _vq_body,
        grid=(NCHUNKS,),
        in_specs=[
            pl.BlockSpec((D, CHUNK), lambda i: (0, i)),
            pl.BlockSpec((NC, D), lambda i: (0, 0)),
        ],
        out_specs=[
            pl.BlockSpec((1, 1, CHUNK), lambda i: (i, 0, 0)),
            pl.BlockSpec((1, 1), lambda i: (0, 0)),
        ],
        out_shape=[
            jax.ShapeDtypeStruct((NCHUNKS, 1, CHUNK), jnp.int32),
            jax.ShapeDtypeStruct((1, 1), f),
        ],
    )(z_flat, codebook)
    indices = idx3.reshape(N)
    commit_loss = loss[0, 0]

    zq_rows = _make_sc_gather()(codebook, indices)

    zq_b = zq_rows.reshape(B, HQ * WQ, D).transpose(0, 2, 1)
    wd = jnp.stack([dec_w1[:, :, kh, kw].T
                    for kh in range(4) for kw in range(4)])
    bd1 = dec_b1.reshape(32, 1)
    w5 = jnp.stack([dec_w2[:, 0, kh, kw]
                    for kh in range(4) for kw in range(4)])
    bd2 = dec_b2.reshape(1, 1)
    xh16 = pl.pallas_call(
        _decoder_body,
        grid=(B,),
        in_specs=[
            pl.BlockSpec((1, D, HQ * WQ), lambda b: (b, 0, 0)),
            pl.BlockSpec((16, 32, D), lambda b: (0, 0, 0)),
            pl.BlockSpec((32, 1), lambda b: (0, 0)),
            pl.BlockSpec((16, 32), lambda b: (0, 0)),
            pl.BlockSpec((1, 1), lambda b: (0, 0)),
        ],
        out_specs=pl.BlockSpec((1, 16, HQ, WQ), lambda b: (b, 0, 0, 0)),
        out_shape=jax.ShapeDtypeStruct((B, 16, HQ, WQ), f),
    )(zq_b, wd, bd1, w5, bd2)
    x_hat = xh16.reshape(B, 4, 4, HQ, WQ).transpose(0, 3, 1, 4, 2)
    x_hat = x_hat.reshape(B, 1, H, W)

    return (x_hat, indices, commit_loss)


# speedup vs baseline: 2.6764x; 2.6764x over previous
import functools

import jax
import jax.numpy as jnp
from jax import lax
from jax.experimental import pallas as pl
from jax.experimental.pallas import tpu as pltpu
from jax.experimental.pallas import tpu_sc as plsc

B = 8
H = W = 512
D = 64
NC = 1024
H1 = W1 = 256
HQ = WQ = 128
N = B * HQ * WQ
CHUNK = 2048
NCHUNKS = N // CHUNK

_f32 = jnp.float32


def _shift_axis(y, axis, d):
    if d == 0:
        return y
    zero_shape = list(y.shape)
    zero_shape[axis] = 1
    z = jnp.zeros(zero_shape, y.dtype)
    sl = [slice(None)] * y.ndim
    if d == -1:
        sl[axis] = slice(0, y.shape[axis] - 1)
        return jnp.concatenate([z, y[tuple(sl)]], axis=axis)
    else:
        sl[axis] = slice(1, None)
        return jnp.concatenate([y[tuple(sl)], z], axis=axis)


def _encoder_body(x_ref, g4r_ref, g4ct_ref, w1_ref, b1_ref, w2_ref,
                  b2_ref, o_ref):
    x = x_ref[0]
    hw = HQ * WQ
    xr = [jnp.dot(g4r_ref[m], x, preferred_element_type=_f32)
          for m in range(4)]
    xg = [[jnp.dot(xr[m], g4ct_ref[n], preferred_element_type=_f32)
           for n in range(4)] for m in range(4)]

    z1p = [[None, None], [None, None]]
    for a in range(2):
        for bb in range(2):
            planes = []
            for kh in range(4):
                v = 2 * a + kh - 1
                m, dr = v % 4, (v - (v % 4)) // 4
                for kw in range(4):
                    u = 2 * bb + kw - 1
                    n, dc = u % 4, (u - (u % 4)) // 4
                    planes.append(_shift_axis(
                        _shift_axis(xg[m][n], 0, dr), 1, dc))
            p = jnp.stack(planes).reshape(16, hw)
            z1 = jnp.dot(w1_ref[...], p, preferred_element_type=_f32)
            z1 = jnp.maximum(z1 + b1_ref[...], 0.0)
            z1p[a][bb] = z1.reshape(32, HQ, WQ)

    taps = ((1, -1), (0, 0), (1, 0), (0, 1))
    acc = jnp.zeros((D, hw), _f32)
    for kh in range(4):
        a, dr = taps[kh]
        parts = [_shift_axis(_shift_axis(z1p[a][taps[kw][0]], 1, dr),
                             2, taps[kw][1]) for kw in range(4)]
        tk = jnp.concatenate(parts, axis=0).reshape(128, hw)
        acc += jnp.dot(w2_ref[kh], tk, preferred_element_type=_f32)
    o_ref[0] = jnp.maximum(acc + b2_ref[...], 0.0)


def _vq_body(z_ref, cb_ref, cbt_ref, idx_ref, zq_ref, loss_ref):
    i = pl.program_id(0)
    z = z_ref[...]
    cb = cb_ref[...]
    s = jnp.dot(cb * -2.0, z, preferred_element_type=_f32)
    cbn = jnp.sum(cb * cb, axis=1, keepdims=True)
    t = s + cbn
    m = jnp.min(t, axis=0, keepdims=True)
    rows = jax.lax.broadcasted_iota(jnp.int32, t.shape, 0)
    idx = jnp.min(jnp.where(t == m, rows, NC), axis=0, keepdims=True)
    idx_ref[0] = idx
    oh = (rows == idx).astype(_f32)
    zq_ref[...] = jnp.dot(cbt_ref[...], oh, preferred_element_type=_f32)
    part = jnp.sum(z * z, keepdims=True) + jnp.sum(m, keepdims=True)

    @pl.when(i == 0)
    def _init():
        loss_ref[...] = jnp.zeros_like(loss_ref)

    loss_ref[...] += part

    @pl.when(i == NCHUNKS - 1)
    def _norm():
        loss_ref[...] *= 1.0 / (N * D)


_NW = 32
_GCH = 256
_BPW = N // _NW
_DP = 128


def _sc_gather_body(cb_hbm, idx_hbm, out_hbm, idx_v, r0, r1, s0, s1):
    wid = lax.axis_index("s") * 2 + lax.axis_index("c")
    base = wid * _BPW
    pltpu.sync_copy(idx_hbm.at[pl.ds(base, _BPW)], idx_v)
    bufs, sems = (r0, r1), (s0, s1)
    nch = _BPW // _GCH
    cps = {0: pltpu.async_copy(cb_hbm.at[idx_v.at[pl.ds(0, _GCH)]], r0, s0)}
    for c in range(nch):
        if c + 1 < nch:
            cps[c + 1] = pltpu.async_copy(
                cb_hbm.at[idx_v.at[pl.ds((c + 1) * _GCH, _GCH)]],
                bufs[(c + 1) % 2], sems[(c + 1) % 2])
        cps[c].wait()
        pltpu.sync_copy(bufs[c % 2], out_hbm.at[pl.ds(base + c * _GCH, _GCH)])


def _make_sc_gather():
    return functools.partial(
        pl.kernel,
        mesh=plsc.VectorSubcoreMesh(core_axis_name="c", subcore_axis_name="s"),
        out_type=jax.ShapeDtypeStruct((N, _DP), jnp.float32),
        scratch_types=[
            pltpu.VMEM((_BPW,), jnp.int32),
            pltpu.VMEM((_GCH, _DP), jnp.float32),
            pltpu.VMEM((_GCH, _DP), jnp.float32),
            pltpu.SemaphoreType.DMA,
            pltpu.SemaphoreType.DMA,
        ],
    )(_sc_gather_body)


_PTAPS = {0: ((1, 0), (3, -1)), 1: ((0, 1), (2, 0))}
_OTAPS = {0: ((1, 0, 0), (3, 1, -1)), 1: ((0, 1, 0), (2, 0, 0)),
          2: ((1, 1, 0), (3, 0, 0)), 3: ((0, 0, 1), (2, 1, 0))}


def _decoder_body(zq_ref, wd_ref, bd1_ref, w5_ref, bd2_ref, o_ref):
    zq = zq_ref[0].reshape(D, HQ, WQ)
    hw = HQ * WQ
    q = [[None, None], [None, None]]
    for py in range(2):
        for px in range(2):
            acc = jnp.zeros((32, hw), _f32)
            for (kh, dy) in _PTAPS[py]:
                for (kw, dx) in _PTAPS[px]:
                    t = _shift_axis(_shift_axis(zq, 1, dy), 2, dx)
                    acc += jnp.dot(wd_ref[kh * 4 + kw], t.reshape(D, hw),
                                   preferred_element_type=_f32)
            d_p = jnp.maximum(acc + bd1_ref[...], 0.0)
            qq = jnp.dot(w5_ref[...], d_p, preferred_element_type=_f32)
            q[py][px] = qq.reshape(16, HQ, WQ)
    for s in range(4):
        for t in range(4):
            acc = jnp.zeros((HQ, WQ), _f32)
            for (kh, rp, dy) in _OTAPS[s]:
                for (kw, cp, dx) in _OTAPS[t]:
                    acc += _shift_axis(
                        _shift_axis(q[rp][cp][kh * 4 + kw], 0, dy), 1, dx)
            o_ref[0, s * 4 + t] = jax.nn.sigmoid(acc + bd2_ref[...])


def kernel(x, enc_w1, enc_b1, enc_w2, enc_b2, codebook,
           dec_w1, dec_b1, dec_w2, dec_b2):
    f = _f32
    xs = x[:, 0]
    r4 = jnp.arange(4, dtype=jnp.int32)
    i128 = jnp.arange(128, dtype=jnp.int32)
    i512 = jnp.arange(512, dtype=jnp.int32)
    g4r = (i512[None, None, :] == 4 * i128[None, :, None]
           + r4[:, None, None]).astype(f)
    g4ct = jnp.swapaxes(g4r, 1, 2)
    w1 = enc_w1.reshape(32, 16)
    b1 = enc_b1.reshape(32, 1)
    w2 = jnp.stack([jnp.concatenate([enc_w2[:, :, k, kw]
                                     for kw in range(4)], axis=1)
                    for k in range(4)])
    b2 = enc_b2.reshape(D, 1)
    z_e = pl.pallas_call(
        _encoder_body,
        grid=(B,),
        in_specs=[
            pl.BlockSpec((1, H, W), lambda b: (b, 0, 0)),
            pl.BlockSpec((4, 128, 512), lambda b: (0, 0, 0)),
            pl.BlockSpec((4, 512, 128), lambda b: (0, 0, 0)),
            pl.BlockSpec((32, 16), lambda b: (0, 0)),
            pl.BlockSpec((32, 1), lambda b: (0, 0)),
            pl.BlockSpec((4, D, 128), lambda b: (0, 0, 0)),
            pl.BlockSpec((D, 1), lambda b: (0, 0)),
        ],
        out_specs=pl.BlockSpec((1, D, HQ * WQ), lambda b: (b, 0, 0)),
        out_shape=jax.ShapeDtypeStruct((B, D, HQ * WQ), f),
    )(xs, g4r, g4ct, w1, b1, w2, b2)

    z_flat = z_e.transpose(1, 0, 2).reshape(D, N)
    idx3, zq, loss = pl.pallas_call(
        _vq_body,
        grid=(NCHUNKS,),
        in_specs=[
            pl.BlockSpec((D, CHUNK), lambda i: (0, i)),
            pl.BlockSpec((NC, D), lambda i: (0, 0)),
            pl.BlockSpec((D, NC), lambda i: (0, 0)),
        ],
        out_specs=[
            pl.BlockSpec((1, 1, CHUNK), lambda i: (i, 0, 0)),
            pl.BlockSpec((D, CHUNK), lambda i: (0, i)),
            pl.BlockSpec((1, 1), lambda i: (0, 0)),
        ],
        out_shape=[
            jax.ShapeDtypeStruct((NCHUNKS, 1, CHUNK), jnp.int32),
            jax.ShapeDtypeStruct((D, N), f),
            jax.ShapeDtypeStruct((1, 1), f),
        ],
    )(z_flat, codebook, codebook.T)
    indices = idx3.reshape(N)
    commit_loss = loss[0, 0]

    zq_b = zq.reshape(D, B, HQ * WQ).transpose(1, 0, 2)
    wd = jnp.stack([dec_w1[:, :, kh, kw].T
                    for kh in range(4) for kw in range(4)])
    bd1 = dec_b1.reshape(32, 1)
    w5 = jnp.stack([dec_w2[:, 0, kh, kw]
                    for kh in range(4) for kw in range(4)])
    bd2 = dec_b2.reshape(1, 1)
    xh16 = pl.pallas_call(
        _decoder_body,
        grid=(B,),
        in_specs=[
            pl.BlockSpec((1, D, HQ * WQ), lambda b: (b, 0, 0)),
            pl.BlockSpec((16, 32, D), lambda b: (0, 0, 0)),
            pl.BlockSpec((32, 1), lambda b: (0, 0)),
            pl.BlockSpec((16, 32), lambda b: (0, 0)),
            pl.BlockSpec((1, 1), lambda b: (0, 0)),
        ],
        out_specs=pl.BlockSpec((1, 16, HQ, WQ), lambda b: (b, 0, 0, 0)),
        out_shape=jax.ShapeDtypeStruct((B, 16, HQ, WQ), f),
    )(zq_b, wd, bd1, w5, bd2)
    x_hat = xh16.reshape(B, 4, 4, HQ, WQ).transpose(0, 3, 1, 4, 2)
    x_hat = x_hat.reshape(B, 1, H, W)

    return (x_hat, indices, commit_loss)


# speedup vs baseline: 2.6901x; 1.0051x over previous
import functools

import jax
import jax.numpy as jnp
from jax import lax
from jax.experimental import pallas as pl
from jax.experimental.pallas import tpu as pltpu
from jax.experimental.pallas import tpu_sc as plsc

B = 8
H = W = 512
D = 64
NC = 1024
H1 = W1 = 256
HQ = WQ = 128
N = B * HQ * WQ
CHUNK = 4096
NCHUNKS = N // CHUNK

_f32 = jnp.float32


def _shift_axis(y, axis, d):
    if d == 0:
        return y
    zero_shape = list(y.shape)
    zero_shape[axis] = 1
    z = jnp.zeros(zero_shape, y.dtype)
    sl = [slice(None)] * y.ndim
    if d == -1:
        sl[axis] = slice(0, y.shape[axis] - 1)
        return jnp.concatenate([z, y[tuple(sl)]], axis=axis)
    else:
        sl[axis] = slice(1, None)
        return jnp.concatenate([y[tuple(sl)], z], axis=axis)


def _encoder_body(x_ref, g4r_ref, g4ct_ref, w1_ref, b1_ref, w2_ref,
                  b2_ref, o_ref):
    x = x_ref[0]
    hw = HQ * WQ
    xr = [jnp.dot(g4r_ref[m], x, preferred_element_type=_f32)
          for m in range(4)]
    xg = [[jnp.dot(xr[m], g4ct_ref[n], preferred_element_type=_f32)
           for n in range(4)] for m in range(4)]

    z1p = [[None, None], [None, None]]
    for a in range(2):
        for bb in range(2):
            planes = []
            for kh in range(4):
                v = 2 * a + kh - 1
                m, dr = v % 4, (v - (v % 4)) // 4
                for kw in range(4):
                    u = 2 * bb + kw - 1
                    n, dc = u % 4, (u - (u % 4)) // 4
                    planes.append(_shift_axis(
                        _shift_axis(xg[m][n], 0, dr), 1, dc))
            p = jnp.stack(planes).reshape(16, hw)
            z1 = jnp.dot(w1_ref[...], p, preferred_element_type=_f32)
            z1 = jnp.maximum(z1 + b1_ref[...], 0.0)
            z1p[a][bb] = z1.reshape(32, HQ, WQ)

    taps = ((1, -1), (0, 0), (1, 0), (0, 1))
    acc = jnp.zeros((D, hw), _f32)
    for kh in range(4):
        a, dr = taps[kh]
        parts = [_shift_axis(_shift_axis(z1p[a][taps[kw][0]], 1, dr),
                             2, taps[kw][1]) for kw in range(4)]
        tk = jnp.concatenate(parts, axis=0).reshape(128, hw)
        acc += jnp.dot(w2_ref[kh], tk, preferred_element_type=_f32)
    o_ref[0] = jnp.maximum(acc + b2_ref[...], 0.0)


def _vq_body(z_ref, cb_ref, cbt_ref, idx_ref, zq_ref, loss_ref):
    i = pl.program_id(0)
    z = z_ref[...]
    cb = cb_ref[...]
    s = jnp.dot(cb * -2.0, z, preferred_element_type=_f32)
    cbn = jnp.sum(cb * cb, axis=1, keepdims=True)
    t = s + cbn
    m = jnp.min(t, axis=0, keepdims=True)
    rows = jax.lax.broadcasted_iota(jnp.int32, t.shape, 0)
    idx = jnp.min(jnp.where(t == m, rows, NC), axis=0, keepdims=True)
    idx_ref[0] = idx
    oh = (rows == idx).astype(_f32)
    zq_ref[...] = jnp.dot(cbt_ref[...], oh, preferred_element_type=_f32)
    part = jnp.sum(z * z, keepdims=True) + jnp.sum(m, keepdims=True)

    @pl.when(i == 0)
    def _init():
        loss_ref[...] = jnp.zeros_like(loss_ref)

    loss_ref[...] += part

    @pl.when(i == NCHUNKS - 1)
    def _norm():
        loss_ref[...] *= 1.0 / (N * D)


_NW = 32
_GCH = 256
_BPW = N // _NW
_DP = 128


def _sc_gather_body(cb_hbm, idx_hbm, out_hbm, idx_v, r0, r1, s0, s1):
    wid = lax.axis_index("s") * 2 + lax.axis_index("c")
    base = wid * _BPW
    pltpu.sync_copy(idx_hbm.at[pl.ds(base, _BPW)], idx_v)
    bufs, sems = (r0, r1), (s0, s1)
    nch = _BPW // _GCH
    cps = {0: pltpu.async_copy(cb_hbm.at[idx_v.at[pl.ds(0, _GCH)]], r0, s0)}
    for c in range(nch):
        if c + 1 < nch:
            cps[c + 1] = pltpu.async_copy(
                cb_hbm.at[idx_v.at[pl.ds((c + 1) * _GCH, _GCH)]],
                bufs[(c + 1) % 2], sems[(c + 1) % 2])
        cps[c].wait()
        pltpu.sync_copy(bufs[c % 2], out_hbm.at[pl.ds(base + c * _GCH, _GCH)])


def _make_sc_gather():
    return functools.partial(
        pl.kernel,
        mesh=plsc.VectorSubcoreMesh(core_axis_name="c", subcore_axis_name="s"),
        out_type=jax.ShapeDtypeStruct((N, _DP), jnp.float32),
        scratch_types=[
            pltpu.VMEM((_BPW,), jnp.int32),
            pltpu.VMEM((_GCH, _DP), jnp.float32),
            pltpu.VMEM((_GCH, _DP), jnp.float32),
            pltpu.SemaphoreType.DMA,
            pltpu.SemaphoreType.DMA,
        ],
    )(_sc_gather_body)


_PTAPS = {0: ((1, 0), (3, -1)), 1: ((0, 1), (2, 0))}
_OTAPS = {0: ((1, 0, 0), (3, 1, -1)), 1: ((0, 1, 0), (2, 0, 0)),
          2: ((1, 1, 0), (3, 0, 0)), 3: ((0, 0, 1), (2, 1, 0))}


def _decoder_body(zq_ref, wd_ref, bd1_ref, w5_ref, bd2_ref, o_ref):
    zq = zq_ref[0].reshape(D, HQ, WQ)
    hw = HQ * WQ
    q = [[None, None], [None, None]]
    for py in range(2):
        for px in range(2):
            acc = jnp.zeros((32, hw), _f32)
            for (kh, dy) in _PTAPS[py]:
                for (kw, dx) in _PTAPS[px]:
                    t = _shift_axis(_shift_axis(zq, 1, dy), 2, dx)
                    acc += jnp.dot(wd_ref[kh * 4 + kw], t.reshape(D, hw),
                                   preferred_element_type=_f32)
            d_p = jnp.maximum(acc + bd1_ref[...], 0.0)
            qq = jnp.dot(w5_ref[...], d_p, preferred_element_type=_f32)
            q[py][px] = qq.reshape(16, HQ, WQ)
    for s in range(4):
        for t in range(4):
            acc = jnp.zeros((HQ, WQ), _f32)
            for (kh, rp, dy) in _OTAPS[s]:
                for (kw, cp, dx) in _OTAPS[t]:
                    acc += _shift_axis(
                        _shift_axis(q[rp][cp][kh * 4 + kw], 0, dy), 1, dx)
            o_ref[0, s * 4 + t] = jax.nn.sigmoid(acc + bd2_ref[...])


def kernel(x, enc_w1, enc_b1, enc_w2, enc_b2, codebook,
           dec_w1, dec_b1, dec_w2, dec_b2):
    f = _f32
    xs = x[:, 0]
    r4 = jnp.arange(4, dtype=jnp.int32)
    i128 = jnp.arange(128, dtype=jnp.int32)
    i512 = jnp.arange(512, dtype=jnp.int32)
    g4r = (i512[None, None, :] == 4 * i128[None, :, None]
           + r4[:, None, None]).astype(f)
    g4ct = jnp.swapaxes(g4r, 1, 2)
    w1 = enc_w1.reshape(32, 16)
    b1 = enc_b1.reshape(32, 1)
    w2 = jnp.stack([jnp.concatenate([enc_w2[:, :, k, kw]
                                     for kw in range(4)], axis=1)
                    for k in range(4)])
    b2 = enc_b2.reshape(D, 1)
    z_e = pl.pallas_call(
        _encoder_body,
        grid=(B,),
        in_specs=[
            pl.BlockSpec((1, H, W), lambda b: (b, 0, 0)),
            pl.BlockSpec((4, 128, 512), lambda b: (0, 0, 0)),
            pl.BlockSpec((4, 512, 128), lambda b: (0, 0, 0)),
            pl.BlockSpec((32, 16), lambda b: (0, 0)),
            pl.BlockSpec((32, 1), lambda b: (0, 0)),
            pl.BlockSpec((4, D, 128), lambda b: (0, 0, 0)),
            pl.BlockSpec((D, 1), lambda b: (0, 0)),
        ],
        out_specs=pl.BlockSpec((1, D, HQ * WQ), lambda b: (b, 0, 0)),
        out_shape=jax.ShapeDtypeStruct((B, D, HQ * WQ), f),
    )(xs, g4r, g4ct, w1, b1, w2, b2)

    z_flat = z_e.transpose(1, 0, 2).reshape(D, N)
    idx3, zq, loss = pl.pallas_call(
        _vq_body,
        grid=(NCHUNKS,),
        in_specs=[
            pl.BlockSpec((D, CHUNK), lambda i: (0, i)),
            pl.BlockSpec((NC, D), lambda i: (0, 0)),
            pl.BlockSpec((D, NC), lambda i: (0, 0)),
        ],
        out_specs=[
            pl.BlockSpec((1, 1, CHUNK), lambda i: (i, 0, 0)),
            pl.BlockSpec((D, CHUNK), lambda i: (0, i)),
            pl.BlockSpec((1, 1), lambda i: (0, 0)),
        ],
        out_shape=[
            jax.ShapeDtypeStruct((NCHUNKS, 1, CHUNK), jnp.int32),
            jax.ShapeDtypeStruct((D, N), f),
            jax.ShapeDtypeStruct((1, 1), f),
        ],
    )(z_flat, codebook, codebook.T)
    indices = idx3.reshape(N)
    commit_loss = loss[0, 0]

    zq_b = zq.reshape(D, B, HQ * WQ).transpose(1, 0, 2)
    wd = jnp.stack([dec_w1[:, :, kh, kw].T
                    for kh in range(4) for kw in range(4)])
    bd1 = dec_b1.reshape(32, 1)
    w5 = jnp.stack([dec_w2[:, 0, kh, kw]
                    for kh in range(4) for kw in range(4)])
    bd2 = dec_b2.reshape(1, 1)
    xh16 = pl.pallas_call(
        _decoder_body,
        grid=(B,),
        in_specs=[
            pl.BlockSpec((1, D, HQ * WQ), lambda b: (b, 0, 0)),
            pl.BlockSpec((16, 32, D), lambda b: (0, 0, 0)),
            pl.BlockSpec((32, 1), lambda b: (0, 0)),
            pl.BlockSpec((16, 32), lambda b: (0, 0)),
            pl.BlockSpec((1, 1), lambda b: (0, 0)),
        ],
        out_specs=pl.BlockSpec((1, 16, HQ, WQ), lambda b: (b, 0, 0, 0)),
        out_shape=jax.ShapeDtypeStruct((B, 16, HQ, WQ), f),
    )(zq_b, wd, bd1, w5, bd2)
    x_hat = xh16.reshape(B, 4, 4, HQ, WQ).transpose(0, 3, 1, 4, 2)
    x_hat = x_hat.reshape(B, 1, H, W)

    return (x_hat, indices, commit_loss)


# speedup vs baseline: 2.6926x; 1.0009x over previous
import jax
import jax.numpy as jnp
from jax.experimental import pallas as pl

B = 8
H = W = 512
D = 64
NC = 1024
H1 = W1 = 256
HQ = WQ = 128
N = B * HQ * WQ
CHUNK = 4096
NCHUNKS = N // CHUNK

_f32 = jnp.float32


def _shift_axis(y, axis, d):
    if d == 0:
        return y
    zero_shape = list(y.shape)
    zero_shape[axis] = 1
    z = jnp.zeros(zero_shape, y.dtype)
    sl = [slice(None)] * y.ndim
    if d == -1:
        sl[axis] = slice(0, y.shape[axis] - 1)
        return jnp.concatenate([z, y[tuple(sl)]], axis=axis)
    else:
        sl[axis] = slice(1, None)
        return jnp.concatenate([y[tuple(sl)], z], axis=axis)


def _encoder_body(x_ref, g4r_ref, g4ct_ref, w1_ref, b1_ref, w2_ref,
                  b2_ref, o_ref):
    x = x_ref[0]
    hw = HQ * WQ
    xr = [jnp.dot(g4r_ref[m], x, preferred_element_type=_f32)
          for m in range(4)]
    xg = [[jnp.dot(xr[m], g4ct_ref[n], preferred_element_type=_f32)
           for n in range(4)] for m in range(4)]

    z1p = [[None, None], [None, None]]
    for a in range(2):
        for bb in range(2):
            planes = []
            for kh in range(4):
                v = 2 * a + kh - 1
                m, dr = v % 4, (v - (v % 4)) // 4
                for kw in range(4):
                    u = 2 * bb + kw - 1
                    n, dc = u % 4, (u - (u % 4)) // 4
                    planes.append(_shift_axis(
                        _shift_axis(xg[m][n], 0, dr), 1, dc))
            p = jnp.stack(planes).reshape(16, hw)
            z1 = jnp.dot(w1_ref[...], p, preferred_element_type=_f32)
            z1 = jnp.maximum(z1 + b1_ref[...], 0.0)
            z1p[a][bb] = z1.reshape(32, HQ, WQ)

    taps = ((1, -1), (0, 0), (1, 0), (0, 1))
    acc = jnp.zeros((D, hw), _f32)
    for kh in range(4):
        a, dr = taps[kh]
        parts = [_shift_axis(_shift_axis(z1p[a][taps[kw][0]], 1, dr),
                             2, taps[kw][1]) for kw in range(4)]
        tk = jnp.concatenate(parts, axis=0).reshape(128, hw)
        acc += jnp.dot(w2_ref[kh], tk, preferred_element_type=_f32)
    o_ref[0] = jnp.maximum(acc + b2_ref[...], 0.0)


def _vq_body(z_ref, cb_ref, cbt_ref, idx_ref, zq_ref, loss_ref):
    i = pl.program_id(0)
    z = z_ref[...]
    cb = cb_ref[...]
    s = jnp.dot(cb * -2.0, z, preferred_element_type=_f32)
    cbn = jnp.sum(cb * cb, axis=1, keepdims=True)
    t = s + cbn
    m = jnp.min(t, axis=0, keepdims=True)
    rows = jax.lax.broadcasted_iota(jnp.int32, t.shape, 0)
    idx = jnp.min(jnp.where(t == m, rows, NC), axis=0, keepdims=True)
    idx_ref[0] = idx
    oh = (rows == idx).astype(_f32)
    zq_ref[...] = jnp.dot(cbt_ref[...], oh, preferred_element_type=_f32)
    part = jnp.sum(z * z, keepdims=True) + jnp.sum(m, keepdims=True)

    @pl.when(i == 0)
    def _init():
        loss_ref[...] = jnp.zeros_like(loss_ref)

    loss_ref[...] += part

    @pl.when(i == NCHUNKS - 1)
    def _norm():
        loss_ref[...] *= 1.0 / (N * D)


_PTAPS = {0: ((1, 0), (3, -1)), 1: ((0, 1), (2, 0))}
_OTAPS = {0: ((1, 0, 0), (3, 1, -1)), 1: ((0, 1, 0), (2, 0, 0)),
          2: ((1, 1, 0), (3, 0, 0)), 3: ((0, 0, 1), (2, 1, 0))}


def _decoder_body(zq_ref, wd_ref, bd1_ref, w5_ref, bd2_ref, o_ref):
    zq = zq_ref[0].reshape(D, HQ, WQ)
    hw = HQ * WQ
    q = [[None, None], [None, None]]
    for py in range(2):
        for px in range(2):
            acc = jnp.zeros((32, hw), _f32)
            for (kh, dy) in _PTAPS[py]:
                for (kw, dx) in _PTAPS[px]:
                    t = _shift_axis(_shift_axis(zq, 1, dy), 2, dx)
                    acc += jnp.dot(wd_ref[kh * 4 + kw], t.reshape(D, hw),
                                   preferred_element_type=_f32)
            d_p = jnp.maximum(acc + bd1_ref[...], 0.0)
            qq = jnp.dot(w5_ref[...], d_p, preferred_element_type=_f32)
            q[py][px] = qq.reshape(16, HQ, WQ)
    for s in range(4):
        for t in range(4):
            acc = jnp.zeros((HQ, WQ), _f32)
            for (kh, rp, dy) in _OTAPS[s]:
                for (kw, cp, dx) in _OTAPS[t]:
                    acc += _shift_axis(
                        _shift_axis(q[rp][cp][kh * 4 + kw], 0, dy), 1, dx)
            o_ref[0, s * 4 + t] = jax.nn.sigmoid(acc + bd2_ref[...])


def kernel(x, enc_w1, enc_b1, enc_w2, enc_b2, codebook,
           dec_w1, dec_b1, dec_w2, dec_b2):
    f = _f32
    xs = x[:, 0]
    r4 = jnp.arange(4, dtype=jnp.int32)
    i128 = jnp.arange(128, dtype=jnp.int32)
    i512 = jnp.arange(512, dtype=jnp.int32)
    g4r = (i512[None, None, :] == 4 * i128[None, :, None]
           + r4[:, None, None]).astype(f)
    g4ct = jnp.swapaxes(g4r, 1, 2)
    w1 = enc_w1.reshape(32, 16)
    b1 = enc_b1.reshape(32, 1)
    w2 = jnp.stack([jnp.concatenate([enc_w2[:, :, k, kw]
                                     for kw in range(4)], axis=1)
                    for k in range(4)])
    b2 = enc_b2.reshape(D, 1)
    z_e = pl.pallas_call(
        _encoder_body,
        grid=(B,),
        in_specs=[
            pl.BlockSpec((1, H, W), lambda b: (b, 0, 0)),
            pl.BlockSpec((4, 128, 512), lambda b: (0, 0, 0)),
            pl.BlockSpec((4, 512, 128), lambda b: (0, 0, 0)),
            pl.BlockSpec((32, 16), lambda b: (0, 0)),
            pl.BlockSpec((32, 1), lambda b: (0, 0)),
            pl.BlockSpec((4, D, 128), lambda b: (0, 0, 0)),
            pl.BlockSpec((D, 1), lambda b: (0, 0)),
        ],
        out_specs=pl.BlockSpec((1, D, HQ * WQ), lambda b: (b, 0, 0)),
        out_shape=jax.ShapeDtypeStruct((B, D, HQ * WQ), f),
    )(xs, g4r, g4ct, w1, b1, w2, b2)

    z_flat = z_e.transpose(1, 0, 2).reshape(D, N)
    idx3, zq, loss = pl.pallas_call(
        _vq_body,
        grid=(NCHUNKS,),
        in_specs=[
            pl.BlockSpec((D, CHUNK), lambda i: (0, i)),
            pl.BlockSpec((NC, D), lambda i: (0, 0)),
            pl.BlockSpec((D, NC), lambda i: (0, 0)),
        ],
        out_specs=[
            pl.BlockSpec((1, 1, CHUNK), lambda i: (i, 0, 0)),
            pl.BlockSpec((D, CHUNK), lambda i: (0, i)),
            pl.BlockSpec((1, 1), lambda i: (0, 0)),
        ],
        out_shape=[
            jax.ShapeDtypeStruct((NCHUNKS, 1, CHUNK), jnp.int32),
            jax.ShapeDtypeStruct((D, N), f),
            jax.ShapeDtypeStruct((1, 1), f),
        ],
    )(z_flat, codebook, codebook.T)
    indices = idx3.reshape(N)
    commit_loss = loss[0, 0]

    zq_b = zq.reshape(D, B, HQ * WQ).transpose(1, 0, 2)
    wd = jnp.stack([dec_w1[:, :, kh, kw].T
                    for kh in range(4) for kw in range(4)])
    bd1 = dec_b1.reshape(32, 1)
    w5 = jnp.stack([dec_w2[:, 0, kh, kw]
                    for kh in range(4) for kw in range(4)])
    bd2 = dec_b2.reshape(1, 1)
    xh16 = pl.pallas_call(
        _decoder_body,
        grid=(B,),
        in_specs=[
            pl.BlockSpec((1, D, HQ * WQ), lambda b: (b, 0, 0)),
            pl.BlockSpec((16, 32, D), lambda b: (0, 0, 0)),
            pl.BlockSpec((32, 1), lambda b: (0, 0)),
            pl.BlockSpec((16, 32), lambda b: (0, 0)),
            pl.BlockSpec((1, 1), lambda b: (0, 0)),
        ],
        out_specs=pl.BlockSpec((1, 16, HQ, WQ), lambda b: (b, 0, 0, 0)),
        out_shape=jax.ShapeDtypeStruct((B, 16, HQ, WQ), f),
    )(zq_b, wd, bd1, w5, bd2)
    x_hat = xh16.reshape(B, 4, 4, HQ, WQ).transpose(0, 3, 1, 4, 2)
    x_hat = x_hat.reshape(B, 1, H, W)

    return (x_hat, indices, commit_loss)
